# baseline (speedup 1.0000x reference)
import functools

import jax
import jax.numpy as jnp
from jax import lax
from jax.experimental import pallas as pl
from jax.experimental.pallas import tpu as pltpu
from jax.experimental.pallas import tpu_sc as plsc

N_NODES = 10000
D = 128
N_EDGES = 320000

NC = 2
NS = 16
NW = NC * NS
EDGES_PER_TILE = N_EDGES // NW
CHUNK = 80
NCHUNK = EDGES_PER_TILE // CHUNK
NPASS = 5
CPASS = NCHUNK // NPASS
N_PAD = 10240
ROWS_PER_TILE = N_PAD // NS


def _mm_body(x_ref, w_ref, o_ref):
    o_ref[...] = jnp.dot(x_ref[...], w_ref[...],
                         preferred_element_type=jnp.float32)


def _matmul(x, w):
    blk = 1000
    return pl.pallas_call(
        _mm_body,
        grid=(N_NODES // blk,),
        in_specs=[pl.BlockSpec((blk, D), lambda i: (i, 0)),
                  pl.BlockSpec((D, D), lambda i: (0, 0))],
        out_specs=pl.BlockSpec((blk, D), lambda i: (i, 0)),
        out_shape=jax.ShapeDtypeStruct((N_NODES, D), jnp.float32),
    )(x, w)


def _comb_mm_body(p_ref, w_ref, o_ref):
    g = jnp.maximum(p_ref[0] + p_ref[1], 0.0)
    o_ref[...] = jnp.dot(g, w_ref[...], preferred_element_type=jnp.float32)


def _comb_matmul(p, w):
    blk = 1000
    return pl.pallas_call(
        _comb_mm_body,
        grid=(N_NODES // blk,),
        in_specs=[pl.BlockSpec((NC, blk, D), lambda i: (0, i, 0)),
                  pl.BlockSpec((D, D), lambda i: (0, 0))],
        out_specs=pl.BlockSpec((blk, D), lambda i: (i, 0)),
        out_shape=jax.ShapeDtypeStruct((N_NODES, D), jnp.float32),
    )(p, w)


def _comb_relu_body(p_ref, o_ref):
    o_ref[...] = jnp.maximum(p_ref[0] + p_ref[1], 0.0)


def _comb_relu(p):
    blk = 1000
    return pl.pallas_call(
        _comb_relu_body,
        grid=(N_NODES // blk,),
        in_specs=[pl.BlockSpec((NC, blk, D), lambda i: (0, i, 0))],
        out_specs=pl.BlockSpec((blk, D), lambda i: (i, 0)),
        out_shape=jax.ShapeDtypeStruct((N_NODES, D), jnp.float32),
    )(p)


def _sc_body(h_hbm, idx_hbm, out_hbm,
             idx_v, rows_v, rows2_v, acc, sem, sem2, sem3, sem4, sem5):
    c = lax.axis_index("c")
    s = lax.axis_index("s")
    wid = c * NS + s

    def src_row(b, j):
        return idx_v.at[b * 2 * CPASS + j]

    def dst_row(b, j):
        return idx_v.at[b * 2 * CPASS + CPASS + j]

    def _zrow(i, carry):
        r = i // 8
        col = (i % 8) * 16
        rows_v[r, pl.ds(col, 16)] = jnp.zeros((16,), jnp.float32)
        return carry
    lax.fori_loop(0, CHUNK * 8, _zrow, 0)
    for t in range(ROWS_PER_TILE // CHUNK):
        pltpu.sync_copy(rows_v, acc.at[pl.ds(s * ROWS_PER_TILE + t * CHUNK,
                                             CHUNK)])

    pltpu.sync_copy(idx_hbm.at[wid, 0], idx_v.at[pl.ds(0, 2 * CPASS)])
    pltpu.async_copy(idx_hbm.at[wid, 1],
                     idx_v.at[pl.ds(2 * CPASS, 2 * CPASS)], sem3)
    pltpu.async_copy(h_hbm.at[src_row(0, 0)], rows_v, sem)
    pltpu.async_copy(h_hbm.at[src_row(0, 1)], rows2_v, sem2)
    plsc.subcore_barrier()

    for p in range(NPASS):
        b = p % 2
        bufA, semA, scA, bufB, semB, scB = (
            (rows_v, sem, sem4, rows2_v, sem2, sem5) if b == 0
            else (rows2_v, sem2, sem5, rows_v, sem, sem4))

        def _pair(i, carry, b=b, bufA=bufA, semA=semA, scA=scA,
                  bufB=bufB, semB=semB, scB=scB):
            j = 2 * i
            pltpu.make_async_copy(h_hbm.at[src_row(b, j)], bufA, semA).wait()
            pltpu.async_copy(bufA, acc.at[dst_row(b, j)], scA, add=True)
            pltpu.make_async_copy(h_hbm.at[src_row(b, j + 1)], bufB,
                                  semB).wait()
            pltpu.async_copy(bufB, acc.at[dst_row(b, j + 1)], scB, add=True)
            pltpu.make_async_copy(bufA, acc.at[dst_row(b, j)], scA).wait()
            pltpu.async_copy(h_hbm.at[src_row(b, j + 2)], bufA, semA)
            pltpu.make_async_copy(bufB, acc.at[dst_row(b, j + 1)], scB).wait()
            pltpu.async_copy(h_hbm.at[src_row(b, j + 3)], bufB, semB)
            return carry

        lax.fori_loop(0, (CPASS - 5) // 2, _pair, 0)
        j = CPASS - 5
        pltpu.make_async_copy(h_hbm.at[src_row(b, j)], bufA, semA).wait()
        pltpu.async_copy(bufA, acc.at[dst_row(b, j)], scA, add=True)
        pltpu.make_async_copy(h_hbm.at[src_row(b, j + 1)], bufB, semB).wait()
        pltpu.async_copy(bufB, acc.at[dst_row(b, j + 1)], scB, add=True)
        pltpu.make_async_copy(bufA, acc.at[dst_row(b, j)], scA).wait()
        pltpu.async_copy(h_hbm.at[src_row(b, j + 2)], bufA, semA)
        pltpu.make_async_copy(bufB, acc.at[dst_row(b, j + 1)], scB).wait()
        pltpu.async_copy(h_hbm.at[src_row(b, j + 3)], bufB, semB)
        pltpu.make_async_copy(h_hbm.at[src_row(b, j + 2)], bufA, semA).wait()
        pltpu.async_copy(bufA, acc.at[dst_row(b, j + 2)], scA, add=True)
        pltpu.make_async_copy(h_hbm.at[src_row(b, j + 3)], bufB, semB).wait()
        pltpu.async_copy(bufB, acc.at[dst_row(b, j + 3)], scB, add=True)
        pltpu.make_async_copy(bufA, acc.at[dst_row(b, j + 2)], scA).wait()
        pltpu.async_copy(h_hbm.at[src_row(b, j + 4)], bufA, semA)
        pltpu.make_async_copy(bufB, acc.at[dst_row(b, j + 3)], scB).wait()
        if p < NPASS - 1:
            pltpu.make_async_copy(idx_hbm.at[wid, p + 1],
                                  idx_v.at[pl.ds((1 - b) * 2 * CPASS,
                                                 2 * CPASS)], sem3).wait()
            pltpu.async_copy(h_hbm.at[src_row(1 - b, 0)], bufB, semB)
        pltpu.make_async_copy(h_hbm.at[src_row(b, j + 4)], bufA, semA).wait()
        pltpu.async_copy(bufA, acc.at[dst_row(b, j + 4)], scA, add=True)
        pltpu.make_async_copy(bufA, acc.at[dst_row(b, j + 4)], scA).wait()
        if p < NPASS - 1:
            pltpu.async_copy(h_hbm.at[src_row(1 - b, 1)], bufA, semA)
        if p < NPASS - 2:
            pltpu.async_copy(idx_hbm.at[wid, p + 2],
                             idx_v.at[pl.ds(b * 2 * CPASS, 2 * CPASS)], sem3)
    plsc.subcore_barrier()

    pltpu.sync_copy(acc.at[pl.ds(s * ROWS_PER_TILE, ROWS_PER_TILE)],
                    out_hbm.at[c, pl.ds(s * ROWS_PER_TILE, ROWS_PER_TILE)])


def _sc_scatter(h, idx4):
    mesh = plsc.VectorSubcoreMesh(core_axis_name="c", subcore_axis_name="s")
    return pl.kernel(
        _sc_body,
        out_type=jax.ShapeDtypeStruct((NC, N_PAD, D), jnp.float32),
        mesh=mesh,
        scratch_types=[
            pltpu.VMEM((4 * CPASS, CHUNK), jnp.int32),
            pltpu.VMEM((CHUNK, D), jnp.float32),
            pltpu.VMEM((CHUNK, D), jnp.float32),
            pltpu.VMEM_SHARED((N_PAD, D), jnp.float32),
            pltpu.SemaphoreType.DMA,
            pltpu.SemaphoreType.DMA,
            pltpu.SemaphoreType.DMA,
            pltpu.SemaphoreType.DMA,
            pltpu.SemaphoreType.DMA,
        ],
    )(h, idx4)


def kernel(x, edge_index, batch, W1, W2):
    src4 = edge_index[0].reshape(NW, NPASS, CPASS, CHUNK)
    dst4 = edge_index[1].reshape(NW, NPASS, CPASS, CHUNK)
    idx4 = jnp.concatenate([src4, dst4], axis=2)
    h1 = _matmul(x, W1)
    p = _sc_scatter(h1, idx4)
    h2 = _comb_matmul(p, W2)
    q = _sc_scatter(h2, idx4)
    return _comb_relu(q)


# speedup vs baseline: 1.2576x; 1.2576x over previous
import functools

import jax
import jax.numpy as jnp
from jax import lax
from jax.experimental import pallas as pl
from jax.experimental.pallas import tpu as pltpu
from jax.experimental.pallas import tpu_sc as plsc

N_NODES = 10000
D = 128
N_EDGES = 320000

NC = 2
NS = 16
NW = NC * NS
EDGES_PER_TILE = N_EDGES // NW
CHUNK = 100
NCHUNK = EDGES_PER_TILE // CHUNK
NPASS = 4
CPASS = NCHUNK // NPASS
N_PAD = 10240
ROWS_PER_TILE = N_PAD // NS


def _mm_body(x_ref, w_ref, o_ref):
    o_ref[...] = jnp.dot(x_ref[...], w_ref[...],
                         preferred_element_type=jnp.float32)


def _matmul(x, w):
    blk = 1000
    return pl.pallas_call(
        _mm_body,
        grid=(N_NODES // blk,),
        in_specs=[pl.BlockSpec((blk, D), lambda i: (i, 0)),
                  pl.BlockSpec((D, D), lambda i: (0, 0))],
        out_specs=pl.BlockSpec((blk, D), lambda i: (i, 0)),
        out_shape=jax.ShapeDtypeStruct((N_NODES, D), jnp.float32),
    )(x, w)


def _comb_mm_body(p_ref, w_ref, o_ref):
    g = jnp.maximum(p_ref[0] + p_ref[1], 0.0)
    o_ref[...] = jnp.dot(g, w_ref[...], preferred_element_type=jnp.float32)


def _comb_matmul(p, w):
    blk = 1000
    return pl.pallas_call(
        _comb_mm_body,
        grid=(N_NODES // blk,),
        in_specs=[pl.BlockSpec((NC, blk, D), lambda i: (0, i, 0)),
                  pl.BlockSpec((D, D), lambda i: (0, 0))],
        out_specs=pl.BlockSpec((blk, D), lambda i: (i, 0)),
        out_shape=jax.ShapeDtypeStruct((N_NODES, D), jnp.float32),
    )(p, w)


def _comb_relu_body(p_ref, o_ref):
    o_ref[...] = jnp.maximum(p_ref[0] + p_ref[1], 0.0)


def _comb_relu(p):
    blk = 1000
    return pl.pallas_call(
        _comb_relu_body,
        grid=(N_NODES // blk,),
        in_specs=[pl.BlockSpec((NC, blk, D), lambda i: (0, i, 0))],
        out_specs=pl.BlockSpec((blk, D), lambda i: (i, 0)),
        out_shape=jax.ShapeDtypeStruct((N_NODES, D), jnp.float32),
    )(p)


def _sc_body(h_hbm, idx_hbm, out_hbm,
             idx_v, rows_v, rows2_v, acc, sem, sem2, sem3):
    c = lax.axis_index("c")
    s = lax.axis_index("s")
    wid = c * NS + s

    def src_row(b, j):
        return idx_v.at[b * 2 * CPASS + j]

    def dst_row(b, j):
        return idx_v.at[b * 2 * CPASS + CPASS + j]

    def _zrow(i, carry):
        r = i // 8
        col = (i % 8) * 16
        rows_v[r, pl.ds(col, 16)] = jnp.zeros((16,), jnp.float32)
        return carry
    lax.fori_loop(0, CHUNK * 8, _zrow, 0)
    for t in range(ROWS_PER_TILE // CHUNK):
        pltpu.sync_copy(rows_v, acc.at[pl.ds(s * ROWS_PER_TILE + t * CHUNK,
                                             CHUNK)])
    _rem = ROWS_PER_TILE - (ROWS_PER_TILE // CHUNK) * CHUNK
    if _rem:
        pltpu.sync_copy(
            rows_v.at[pl.ds(0, _rem)],
            acc.at[pl.ds(s * ROWS_PER_TILE + ROWS_PER_TILE - _rem, _rem)])

    pltpu.sync_copy(idx_hbm.at[wid, 0], idx_v.at[pl.ds(0, 2 * CPASS)])
    pltpu.async_copy(idx_hbm.at[wid, 1],
                     idx_v.at[pl.ds(2 * CPASS, 2 * CPASS)], sem3)
    pltpu.async_copy(h_hbm.at[src_row(0, 0)], rows_v, sem)
    pltpu.async_copy(h_hbm.at[src_row(0, 1)], rows2_v, sem2)
    plsc.subcore_barrier()

    for p in range(NPASS):
        b = p % 2
        bufA, semA, bufB, semB = ((rows_v, sem, rows2_v, sem2) if b == 0
                                  else (rows2_v, sem2, rows_v, sem))

        def _pair(i, carry, b=b, bufA=bufA, semA=semA, bufB=bufB, semB=semB):
            j = 2 * i
            pltpu.make_async_copy(h_hbm.at[src_row(b, j)], bufA, semA).wait()
            pltpu.sync_copy(bufA, acc.at[dst_row(b, j)], add=True)
            pltpu.async_copy(h_hbm.at[src_row(b, j + 2)], bufA, semA)
            pltpu.make_async_copy(h_hbm.at[src_row(b, j + 1)], bufB,
                                  semB).wait()
            pltpu.sync_copy(bufB, acc.at[dst_row(b, j + 1)], add=True)
            pltpu.async_copy(h_hbm.at[src_row(b, j + 3)], bufB, semB)
            return carry

        lax.fori_loop(0, (CPASS - 5) // 2, _pair, 0)
        j = CPASS - 5
        pltpu.make_async_copy(h_hbm.at[src_row(b, j)], bufA, semA).wait()
        pltpu.sync_copy(bufA, acc.at[dst_row(b, j)], add=True)
        pltpu.async_copy(h_hbm.at[src_row(b, j + 2)], bufA, semA)
        pltpu.make_async_copy(h_hbm.at[src_row(b, j + 1)], bufB, semB).wait()
        pltpu.sync_copy(bufB, acc.at[dst_row(b, j + 1)], add=True)
        pltpu.async_copy(h_hbm.at[src_row(b, j + 3)], bufB, semB)
        pltpu.make_async_copy(h_hbm.at[src_row(b, j + 2)], bufA, semA).wait()
        pltpu.sync_copy(bufA, acc.at[dst_row(b, j + 2)], add=True)
        pltpu.async_copy(h_hbm.at[src_row(b, j + 4)], bufA, semA)
        pltpu.make_async_copy(h_hbm.at[src_row(b, j + 3)], bufB, semB).wait()
        pltpu.sync_copy(bufB, acc.at[dst_row(b, j + 3)], add=True)
        if p < NPASS - 1:
            pltpu.make_async_copy(idx_hbm.at[wid, p + 1],
                                  idx_v.at[pl.ds((1 - b) * 2 * CPASS,
                                                 2 * CPASS)], sem3).wait()
            pltpu.async_copy(h_hbm.at[src_row(1 - b, 0)], bufB, semB)
        pltpu.make_async_copy(h_hbm.at[src_row(b, j + 4)], bufA, semA).wait()
        pltpu.sync_copy(bufA, acc.at[dst_row(b, j + 4)], add=True)
        if p < NPASS - 1:
            pltpu.async_copy(h_hbm.at[src_row(1 - b, 1)], bufA, semA)
        if p < NPASS - 2:
            pltpu.async_copy(idx_hbm.at[wid, p + 2],
                             idx_v.at[pl.ds(b * 2 * CPASS, 2 * CPASS)], sem3)
    plsc.subcore_barrier()

    pltpu.sync_copy(acc.at[pl.ds(s * ROWS_PER_TILE, ROWS_PER_TILE)],
                    out_hbm.at[c, pl.ds(s * ROWS_PER_TILE, ROWS_PER_TILE)])


def _sc_scatter(h, idx4):
    mesh = plsc.VectorSubcoreMesh(core_axis_name="c", subcore_axis_name="s")
    return pl.kernel(
        _sc_body,
        out_type=jax.ShapeDtypeStruct((NC, N_PAD, D), jnp.float32),
        mesh=mesh,
        scratch_types=[
            pltpu.VMEM((4 * CPASS, CHUNK), jnp.int32),
            pltpu.VMEM((CHUNK, D), jnp.float32),
            pltpu.VMEM((CHUNK, D), jnp.float32),
            pltpu.VMEM_SHARED((N_PAD, D), jnp.float32),
            pltpu.SemaphoreType.DMA,
            pltpu.SemaphoreType.DMA,
            pltpu.SemaphoreType.DMA,
        ],
    )(h, idx4)


def kernel(x, edge_index, batch, W1, W2):
    src4 = edge_index[0].reshape(NW, NPASS, CPASS, CHUNK)
    dst4 = edge_index[1].reshape(NW, NPASS, CPASS, CHUNK)
    idx4 = jnp.concatenate([src4, dst4], axis=2)
    h1 = _matmul(x, W1)
    p = _sc_scatter(h1, idx4)
    h2 = _comb_matmul(p, W2)
    q = _sc_scatter(h2, idx4)
    return _comb_relu(q)


# speedup vs baseline: 1.2969x; 1.0312x over previous
import functools

import jax
import jax.numpy as jnp
from jax import lax
from jax.experimental import pallas as pl
from jax.experimental.pallas import tpu as pltpu
from jax.experimental.pallas import tpu_sc as plsc

N_NODES = 10000
D = 128
N_EDGES = 320000

NC = 2
NS = 16
NW = NC * NS
EDGES_PER_TILE = N_EDGES // NW
CHUNK = 100
NCHUNK = EDGES_PER_TILE // CHUNK
NPASS = 4
CPASS = NCHUNK // NPASS
N_PAD = 10240
ROWS_PER_TILE = N_PAD // NS


def _comb_mm_relu_body(p_ref, w_ref, o_ref):
    t = p_ref[0] + p_ref[1]
    o_ref[...] = jnp.maximum(
        jnp.dot(t, w_ref[...], preferred_element_type=jnp.float32), 0.0)


def _comb_mm_relu(p, w):
    blk = 1000
    return pl.pallas_call(
        _comb_mm_relu_body,
        grid=(N_NODES // blk,),
        in_specs=[pl.BlockSpec((NC, blk, D), lambda i: (0, i, 0)),
                  pl.BlockSpec((D, D), lambda i: (0, 0))],
        out_specs=pl.BlockSpec((blk, D), lambda i: (i, 0)),
        out_shape=jax.ShapeDtypeStruct((N_NODES, D), jnp.float32),
    )(p, w)


def _sc_body(h_hbm, idx_hbm, out_hbm,
             idx_v, rows_v, rows2_v, acc, sem, sem2, sem3):
    c = lax.axis_index("c")
    s = lax.axis_index("s")
    wid = c * NS + s

    def src_row(b, j):
        return idx_v.at[b * 2 * CPASS + j]

    def dst_row(b, j):
        return idx_v.at[b * 2 * CPASS + CPASS + j]

    def _zrow(i, carry):
        r = i // 8
        col = (i % 8) * 16
        rows_v[r, pl.ds(col, 16)] = jnp.zeros((16,), jnp.float32)
        return carry
    lax.fori_loop(0, CHUNK * 8, _zrow, 0)
    for t in range(ROWS_PER_TILE // CHUNK):
        pltpu.sync_copy(rows_v, acc.at[pl.ds(s * ROWS_PER_TILE + t * CHUNK,
                                             CHUNK)])
    _rem = ROWS_PER_TILE - (ROWS_PER_TILE // CHUNK) * CHUNK
    if _rem:
        pltpu.sync_copy(
            rows_v.at[pl.ds(0, _rem)],
            acc.at[pl.ds(s * ROWS_PER_TILE + ROWS_PER_TILE - _rem, _rem)])

    pltpu.sync_copy(idx_hbm.at[wid, 0], idx_v.at[pl.ds(0, 2 * CPASS)])
    pltpu.async_copy(idx_hbm.at[wid, 1],
                     idx_v.at[pl.ds(2 * CPASS, 2 * CPASS)], sem3)
    pltpu.async_copy(h_hbm.at[src_row(0, 0)], rows_v, sem)
    pltpu.async_copy(h_hbm.at[src_row(0, 1)], rows2_v, sem2)
    plsc.subcore_barrier()

    for p in range(NPASS):
        b = p % 2
        bufA, semA, bufB, semB = ((rows_v, sem, rows2_v, sem2) if b == 0
                                  else (rows2_v, sem2, rows_v, sem))

        def _pair(i, carry, b=b, bufA=bufA, semA=semA, bufB=bufB, semB=semB):
            j = 2 * i
            pltpu.make_async_copy(h_hbm.at[src_row(b, j)], bufA, semA).wait()
            pltpu.sync_copy(bufA, acc.at[dst_row(b, j)], add=True)
            pltpu.async_copy(h_hbm.at[src_row(b, j + 2)], bufA, semA)
            pltpu.make_async_copy(h_hbm.at[src_row(b, j + 1)], bufB,
                                  semB).wait()
            pltpu.sync_copy(bufB, acc.at[dst_row(b, j + 1)], add=True)
            pltpu.async_copy(h_hbm.at[src_row(b, j + 3)], bufB, semB)
            return carry

        lax.fori_loop(0, (CPASS - 5) // 2, _pair, 0)
        j = CPASS - 5
        pltpu.make_async_copy(h_hbm.at[src_row(b, j)], bufA, semA).wait()
        pltpu.sync_copy(bufA, acc.at[dst_row(b, j)], add=True)
        pltpu.async_copy(h_hbm.at[src_row(b, j + 2)], bufA, semA)
        pltpu.make_async_copy(h_hbm.at[src_row(b, j + 1)], bufB, semB).wait()
        pltpu.sync_copy(bufB, acc.at[dst_row(b, j + 1)], add=True)
        pltpu.async_copy(h_hbm.at[src_row(b, j + 3)], bufB, semB)
        pltpu.make_async_copy(h_hbm.at[src_row(b, j + 2)], bufA, semA).wait()
        pltpu.sync_copy(bufA, acc.at[dst_row(b, j + 2)], add=True)
        pltpu.async_copy(h_hbm.at[src_row(b, j + 4)], bufA, semA)
        pltpu.make_async_copy(h_hbm.at[src_row(b, j + 3)], bufB, semB).wait()
        pltpu.sync_copy(bufB, acc.at[dst_row(b, j + 3)], add=True)
        if p < NPASS - 1:
            pltpu.make_async_copy(idx_hbm.at[wid, p + 1],
                                  idx_v.at[pl.ds((1 - b) * 2 * CPASS,
                                                 2 * CPASS)], sem3).wait()
            pltpu.async_copy(h_hbm.at[src_row(1 - b, 0)], bufB, semB)
        pltpu.make_async_copy(h_hbm.at[src_row(b, j + 4)], bufA, semA).wait()
        pltpu.sync_copy(bufA, acc.at[dst_row(b, j + 4)], add=True)
        if p < NPASS - 1:
            pltpu.async_copy(h_hbm.at[src_row(1 - b, 1)], bufA, semA)
        if p < NPASS - 2:
            pltpu.async_copy(idx_hbm.at[wid, p + 2],
                             idx_v.at[pl.ds(b * 2 * CPASS, 2 * CPASS)], sem3)
    plsc.subcore_barrier()

    pltpu.sync_copy(acc.at[pl.ds(s * ROWS_PER_TILE, ROWS_PER_TILE)],
                    out_hbm.at[c, pl.ds(s * ROWS_PER_TILE, ROWS_PER_TILE)])


def _sc_scatter(h, idx4):
    mesh = plsc.VectorSubcoreMesh(core_axis_name="c", subcore_axis_name="s")
    return pl.kernel(
        _sc_body,
        out_type=jax.ShapeDtypeStruct((NC, N_PAD, D), jnp.float32),
        mesh=mesh,
        scratch_types=[
            pltpu.VMEM((4 * CPASS, CHUNK), jnp.int32),
            pltpu.VMEM((CHUNK, D), jnp.float32),
            pltpu.VMEM((CHUNK, D), jnp.float32),
            pltpu.VMEM_SHARED((N_PAD, D), jnp.float32),
            pltpu.SemaphoreType.DMA,
            pltpu.SemaphoreType.DMA,
            pltpu.SemaphoreType.DMA,
        ],
    )(h, idx4)


def kernel(x, edge_index, batch, W1, W2):
    src4 = edge_index[0].reshape(NW, NPASS, CPASS, CHUNK)
    dst4 = edge_index[1].reshape(NW, NPASS, CPASS, CHUNK)
    idx4 = jnp.concatenate([src4, dst4], axis=2)
    p = _sc_scatter(x, idx4)
    g1 = _comb_mm_relu(p, W1)
    q = _sc_scatter(g1, idx4)
    return _comb_mm_relu(q, W2)


# speedup vs baseline: 1.3355x; 1.0298x over previous
import functools

import jax
import jax.numpy as jnp
from jax import lax
from jax.experimental import pallas as pl
from jax.experimental.pallas import tpu as pltpu
from jax.experimental.pallas import tpu_sc as plsc

N_NODES = 10000
D = 128
N_EDGES = 320000

NC = 2
NS = 16
NW = NC * NS
EDGES_PER_TILE = N_EDGES // NW
CHUNK = 125
NCHUNK = EDGES_PER_TILE // CHUNK
NPASS = 16
CPASS = NCHUNK // NPASS
N_PAD = 10240
ROWS_PER_TILE = N_PAD // NS


def _comb_mm_relu_body(p_ref, w_ref, o_ref):
    t = p_ref[0] + p_ref[1]
    o_ref[...] = jnp.maximum(
        jnp.dot(t, w_ref[...], preferred_element_type=jnp.float32), 0.0)


def _comb_mm_relu(p, w):
    blk = 1000
    return pl.pallas_call(
        _comb_mm_relu_body,
        grid=(N_NODES // blk,),
        in_specs=[pl.BlockSpec((NC, blk, D), lambda i: (0, i, 0)),
                  pl.BlockSpec((D, D), lambda i: (0, 0))],
        out_specs=pl.BlockSpec((blk, D), lambda i: (i, 0)),
        out_shape=jax.ShapeDtypeStruct((N_NODES, D), jnp.float32),
    )(p, w)


def _sc_body(h_hbm, idx_hbm, out_hbm,
             idx_v, rows_v, rows2_v, acc, sem, sem2, sem3):
    c = lax.axis_index("c")
    s = lax.axis_index("s")
    wid = c * NS + s

    def src_row(b, j):
        return idx_v.at[b * 2 * CPASS + j]

    def dst_row(b, j):
        return idx_v.at[b * 2 * CPASS + CPASS + j]

    def _zrow(i, carry):
        r = i // 8
        col = (i % 8) * 16
        rows_v[r, pl.ds(col, 16)] = jnp.zeros((16,), jnp.float32)
        return carry
    lax.fori_loop(0, CHUNK * 8, _zrow, 0)
    for t in range(ROWS_PER_TILE // CHUNK):
        pltpu.sync_copy(rows_v, acc.at[pl.ds(s * ROWS_PER_TILE + t * CHUNK,
                                             CHUNK)])
    _rem = ROWS_PER_TILE - (ROWS_PER_TILE // CHUNK) * CHUNK
    if _rem:
        pltpu.sync_copy(
            rows_v.at[pl.ds(0, _rem)],
            acc.at[pl.ds(s * ROWS_PER_TILE + ROWS_PER_TILE - _rem, _rem)])

    pltpu.sync_copy(idx_hbm.at[wid, 0], idx_v.at[pl.ds(0, 2 * CPASS)])
    pltpu.async_copy(idx_hbm.at[wid, 1],
                     idx_v.at[pl.ds(2 * CPASS, 2 * CPASS)], sem3)
    pltpu.async_copy(h_hbm.at[src_row(0, 0)], rows_v, sem)
    pltpu.async_copy(h_hbm.at[src_row(0, 1)], rows2_v, sem2)
    plsc.subcore_barrier()

    for p in range(NPASS):
        b = p % 2
        bufA, semA, bufB, semB = ((rows_v, sem, rows2_v, sem2) if b == 0
                                  else (rows2_v, sem2, rows_v, sem))

        def _pair(i, carry, b=b, bufA=bufA, semA=semA, bufB=bufB, semB=semB):
            j = 2 * i
            pltpu.make_async_copy(h_hbm.at[src_row(b, j)], bufA, semA).wait()
            pltpu.sync_copy(bufA, acc.at[dst_row(b, j)], add=True)
            pltpu.async_copy(h_hbm.at[src_row(b, j + 2)], bufA, semA)
            pltpu.make_async_copy(h_hbm.at[src_row(b, j + 1)], bufB,
                                  semB).wait()
            pltpu.sync_copy(bufB, acc.at[dst_row(b, j + 1)], add=True)
            pltpu.async_copy(h_hbm.at[src_row(b, j + 3)], bufB, semB)
            return carry

        lax.fori_loop(0, (CPASS - 5) // 2, _pair, 0)
        j = CPASS - 5
        pltpu.make_async_copy(h_hbm.at[src_row(b, j)], bufA, semA).wait()
        pltpu.sync_copy(bufA, acc.at[dst_row(b, j)], add=True)
        pltpu.async_copy(h_hbm.at[src_row(b, j + 2)], bufA, semA)
        pltpu.make_async_copy(h_hbm.at[src_row(b, j + 1)], bufB, semB).wait()
        pltpu.sync_copy(bufB, acc.at[dst_row(b, j + 1)], add=True)
        pltpu.async_copy(h_hbm.at[src_row(b, j + 3)], bufB, semB)
        pltpu.make_async_copy(h_hbm.at[src_row(b, j + 2)], bufA, semA).wait()
        pltpu.sync_copy(bufA, acc.at[dst_row(b, j + 2)], add=True)
        pltpu.async_copy(h_hbm.at[src_row(b, j + 4)], bufA, semA)
        pltpu.make_async_copy(h_hbm.at[src_row(b, j + 3)], bufB, semB).wait()
        pltpu.sync_copy(bufB, acc.at[dst_row(b, j + 3)], add=True)
        if p < NPASS - 1:
            pltpu.make_async_copy(idx_hbm.at[wid, p + 1],
                                  idx_v.at[pl.ds((1 - b) * 2 * CPASS,
                                                 2 * CPASS)], sem3).wait()
            pltpu.async_copy(h_hbm.at[src_row(1 - b, 0)], bufB, semB)
        pltpu.make_async_copy(h_hbm.at[src_row(b, j + 4)], bufA, semA).wait()
        pltpu.sync_copy(bufA, acc.at[dst_row(b, j + 4)], add=True)
        if p < NPASS - 1:
            pltpu.async_copy(h_hbm.at[src_row(1 - b, 1)], bufA, semA)
        if p < NPASS - 2:
            pltpu.async_copy(idx_hbm.at[wid, p + 2],
                             idx_v.at[pl.ds(b * 2 * CPASS, 2 * CPASS)], sem3)
    plsc.subcore_barrier()

    pltpu.sync_copy(acc.at[pl.ds(s * ROWS_PER_TILE, ROWS_PER_TILE)],
                    out_hbm.at[c, pl.ds(s * ROWS_PER_TILE, ROWS_PER_TILE)])


def _sc_scatter(h, idx4):
    mesh = plsc.VectorSubcoreMesh(core_axis_name="c", subcore_axis_name="s")
    return pl.kernel(
        _sc_body,
        out_type=jax.ShapeDtypeStruct((NC, N_PAD, D), jnp.float32),
        mesh=mesh,
        scratch_types=[
            pltpu.VMEM((4 * CPASS, CHUNK), jnp.int32),
            pltpu.VMEM((CHUNK, D), jnp.float32),
            pltpu.VMEM((CHUNK, D), jnp.float32),
            pltpu.VMEM_SHARED((N_PAD, D), jnp.float32),
            pltpu.SemaphoreType.DMA,
            pltpu.SemaphoreType.DMA,
            pltpu.SemaphoreType.DMA,
        ],
    )(h, idx4)


def kernel(x, edge_index, batch, W1, W2):
    src4 = edge_index[0].reshape(NW, NPASS, CPASS, CHUNK)
    dst4 = edge_index[1].reshape(NW, NPASS, CPASS, CHUNK)
    idx4 = jnp.concatenate([src4, dst4], axis=2)
    p = _sc_scatter(x, idx4)
    g1 = _comb_mm_relu(p, W1)
    q = _sc_scatter(g1, idx4)
    return _comb_mm_relu(q, W2)


# speedup vs baseline: 1.3823x; 1.0350x over previous
import functools

import jax
import jax.numpy as jnp
from jax import lax
from jax.experimental import pallas as pl
from jax.experimental.pallas import tpu as pltpu
from jax.experimental.pallas import tpu_sc as plsc

N_NODES = 10000
D = 128
N_EDGES = 320000

NC = 2
NS = 16
NW = NC * NS
EDGES_PER_TILE = N_EDGES // NW
CHUNK = 125
NCHUNK = EDGES_PER_TILE // CHUNK
NPASS = 16
CPASS = NCHUNK // NPASS
N_PAD = 10240
ROWS_PER_TILE = N_PAD // NS


def _comb_mm_relu_body(p_ref, w_ref, o_ref):
    t = p_ref[0] + p_ref[1]
    o_ref[...] = jnp.maximum(
        jnp.dot(t, w_ref[...], preferred_element_type=jnp.float32), 0.0)


def _comb_mm_relu(p, w):
    blk = 2000
    return pl.pallas_call(
        _comb_mm_relu_body,
        grid=(N_NODES // blk,),
        in_specs=[pl.BlockSpec((NC, blk, D), lambda i: (0, i, 0)),
                  pl.BlockSpec((D, D), lambda i: (0, 0))],
        out_specs=pl.BlockSpec((blk, D), lambda i: (i, 0)),
        out_shape=jax.ShapeDtypeStruct((N_NODES, D), jnp.float32),
    )(p, w)


def _sc_body(h_hbm, idx_hbm, out_hbm,
             idx_v, rows_v, rows2_v, acc, sem, sem2, sem3):
    c = lax.axis_index("c")
    s = lax.axis_index("s")
    wid = c * NS + s

    def src_row(b, j):
        return idx_v.at[b * 2 * CPASS + j]

    def dst_row(b, j):
        return idx_v.at[b * 2 * CPASS + CPASS + j]

    def _zrow(i, carry):
        r = i // 8
        col = (i % 8) * 16
        rows2_v[r, pl.ds(col, 16)] = jnp.zeros((16,), jnp.float32)
        return carry
    lax.fori_loop(0, CHUNK * 8, _zrow, 0)

    pltpu.sync_copy(idx_hbm.at[wid, 0], idx_v.at[pl.ds(0, 2 * CPASS)])
    pltpu.async_copy(h_hbm.at[src_row(0, 0)], rows_v, sem)
    pltpu.async_copy(idx_hbm.at[wid, 1],
                     idx_v.at[pl.ds(2 * CPASS, 2 * CPASS)], sem3)

    for t in range(ROWS_PER_TILE // CHUNK):
        pltpu.sync_copy(rows2_v, acc.at[pl.ds(s * ROWS_PER_TILE + t * CHUNK,
                                              CHUNK)])
    _rem = ROWS_PER_TILE - (ROWS_PER_TILE // CHUNK) * CHUNK
    if _rem:
        pltpu.sync_copy(
            rows2_v.at[pl.ds(0, _rem)],
            acc.at[pl.ds(s * ROWS_PER_TILE + ROWS_PER_TILE - _rem, _rem)])

    pltpu.async_copy(h_hbm.at[src_row(0, 1)], rows2_v, sem2)
    plsc.subcore_barrier()

    for p in range(NPASS):
        b = p % 2
        bufA, semA, bufB, semB = ((rows_v, sem, rows2_v, sem2) if b == 0
                                  else (rows2_v, sem2, rows_v, sem))

        def _pair(i, carry, b=b, bufA=bufA, semA=semA, bufB=bufB, semB=semB):
            j = 2 * i
            pltpu.make_async_copy(h_hbm.at[src_row(b, j)], bufA, semA).wait()
            pltpu.sync_copy(bufA, acc.at[dst_row(b, j)], add=True)
            pltpu.async_copy(h_hbm.at[src_row(b, j + 2)], bufA, semA)
            pltpu.make_async_copy(h_hbm.at[src_row(b, j + 1)], bufB,
                                  semB).wait()
            pltpu.sync_copy(bufB, acc.at[dst_row(b, j + 1)], add=True)
            pltpu.async_copy(h_hbm.at[src_row(b, j + 3)], bufB, semB)
            return carry

        lax.fori_loop(0, (CPASS - 5) // 2, _pair, 0)
        j = CPASS - 5
        pltpu.make_async_copy(h_hbm.at[src_row(b, j)], bufA, semA).wait()
        pltpu.sync_copy(bufA, acc.at[dst_row(b, j)], add=True)
        pltpu.async_copy(h_hbm.at[src_row(b, j + 2)], bufA, semA)
        pltpu.make_async_copy(h_hbm.at[src_row(b, j + 1)], bufB, semB).wait()
        pltpu.sync_copy(bufB, acc.at[dst_row(b, j + 1)], add=True)
        pltpu.async_copy(h_hbm.at[src_row(b, j + 3)], bufB, semB)
        pltpu.make_async_copy(h_hbm.at[src_row(b, j + 2)], bufA, semA).wait()
        pltpu.sync_copy(bufA, acc.at[dst_row(b, j + 2)], add=True)
        pltpu.async_copy(h_hbm.at[src_row(b, j + 4)], bufA, semA)
        pltpu.make_async_copy(h_hbm.at[src_row(b, j + 3)], bufB, semB).wait()
        pltpu.sync_copy(bufB, acc.at[dst_row(b, j + 3)], add=True)
        if p < NPASS - 1:
            pltpu.make_async_copy(idx_hbm.at[wid, p + 1],
                                  idx_v.at[pl.ds((1 - b) * 2 * CPASS,
                                                 2 * CPASS)], sem3).wait()
            pltpu.async_copy(h_hbm.at[src_row(1 - b, 0)], bufB, semB)
        pltpu.make_async_copy(h_hbm.at[src_row(b, j + 4)], bufA, semA).wait()
        pltpu.sync_copy(bufA, acc.at[dst_row(b, j + 4)], add=True)
        if p < NPASS - 1:
            pltpu.async_copy(h_hbm.at[src_row(1 - b, 1)], bufA, semA)
        if p < NPASS - 2:
            pltpu.async_copy(idx_hbm.at[wid, p + 2],
                             idx_v.at[pl.ds(b * 2 * CPASS, 2 * CPASS)], sem3)
    plsc.subcore_barrier()

    pltpu.sync_copy(acc.at[pl.ds(s * ROWS_PER_TILE, ROWS_PER_TILE)],
                    out_hbm.at[c, pl.ds(s * ROWS_PER_TILE, ROWS_PER_TILE)])


def _sc_scatter(h, idx4):
    mesh = plsc.VectorSubcoreMesh(core_axis_name="c", subcore_axis_name="s")
    return pl.kernel(
        _sc_body,
        out_type=jax.ShapeDtypeStruct((NC, N_PAD, D), jnp.float32),
        mesh=mesh,
        scratch_types=[
            pltpu.VMEM((4 * CPASS, CHUNK), jnp.int32),
            pltpu.VMEM((CHUNK, D), jnp.float32),
            pltpu.VMEM((CHUNK, D), jnp.float32),
            pltpu.VMEM_SHARED((N_PAD, D), jnp.float32),
            pltpu.SemaphoreType.DMA,
            pltpu.SemaphoreType.DMA,
            pltpu.SemaphoreType.DMA,
        ],
    )(h, idx4)


def kernel(x, edge_index, batch, W1, W2):
    src4 = edge_index[0].reshape(NW, NPASS, CPASS, CHUNK)
    dst4 = edge_index[1].reshape(NW, NPASS, CPASS, CHUNK)
    idx4 = jnp.concatenate([src4, dst4], axis=2)
    p = _sc_scatter(x, idx4)
    g1 = _comb_mm_relu(p, W1)
    q = _sc_scatter(g1, idx4)
    return _comb_mm_relu(q, W2)


# speedup vs baseline: 1.4023x; 1.0145x over previous
import functools

import jax
import jax.numpy as jnp
from jax import lax
from jax.experimental import pallas as pl
from jax.experimental.pallas import tpu as pltpu
from jax.experimental.pallas import tpu_sc as plsc

N_NODES = 10000
D = 128
N_EDGES = 320000

NC = 2
NS = 16
NW = NC * NS
EDGES_PER_TILE = N_EDGES // NW
CHUNK = 125
NCHUNK = EDGES_PER_TILE // CHUNK
NPASS = 16
CPASS = NCHUNK // NPASS
N_PAD = 10240
ROWS_PER_TILE = N_PAD // NS


def _comb_mm_relu_body(p_ref, w_ref, o_ref):
    t = p_ref[0] + p_ref[1]
    o_ref[...] = jnp.maximum(
        jnp.dot(t, w_ref[...], preferred_element_type=jnp.float32), 0.0)


def _comb_mm_relu(p, w):
    blk = 5000
    return pl.pallas_call(
        _comb_mm_relu_body,
        grid=(N_NODES // blk,),
        in_specs=[pl.BlockSpec((NC, blk, D), lambda i: (0, i, 0)),
                  pl.BlockSpec((D, D), lambda i: (0, 0))],
        out_specs=pl.BlockSpec((blk, D), lambda i: (i, 0)),
        out_shape=jax.ShapeDtypeStruct((N_NODES, D), jnp.float32),
    )(p, w)


def _sc_body(h_hbm, idx_hbm, out_hbm,
             idx_v, rows_v, rows2_v, acc, sem, sem2, sem3):
    c = lax.axis_index("c")
    s = lax.axis_index("s")
    wid = c * NS + s

    def src_row(b, j):
        return idx_v.at[b * 2 * CPASS + j]

    def dst_row(b, j):
        return idx_v.at[b * 2 * CPASS + CPASS + j]

    def _zrow(i, carry):
        r = i // 8
        col = (i % 8) * 16
        rows2_v[r, pl.ds(col, 16)] = jnp.zeros((16,), jnp.float32)
        return carry
    lax.fori_loop(0, CHUNK * 8, _zrow, 0)

    pltpu.sync_copy(idx_hbm.at[wid, 0], idx_v.at[pl.ds(0, 2 * CPASS)])
    pltpu.async_copy(h_hbm.at[src_row(0, 0)], rows_v, sem)
    pltpu.async_copy(idx_hbm.at[wid, 1],
                     idx_v.at[pl.ds(2 * CPASS, 2 * CPASS)], sem3)

    for t in range(ROWS_PER_TILE // CHUNK):
        pltpu.sync_copy(rows2_v, acc.at[pl.ds(s * ROWS_PER_TILE + t * CHUNK,
                                              CHUNK)])
    _rem = ROWS_PER_TILE - (ROWS_PER_TILE // CHUNK) * CHUNK
    if _rem:
        pltpu.sync_copy(
            rows2_v.at[pl.ds(0, _rem)],
            acc.at[pl.ds(s * ROWS_PER_TILE + ROWS_PER_TILE - _rem, _rem)])

    pltpu.async_copy(h_hbm.at[src_row(0, 1)], rows2_v, sem2)
    plsc.subcore_barrier()

    for p in range(NPASS):
        b = p % 2
        bufA, semA, bufB, semB = ((rows_v, sem, rows2_v, sem2) if b == 0
                                  else (rows2_v, sem2, rows_v, sem))

        def _pair(i, carry, b=b, bufA=bufA, semA=semA, bufB=bufB, semB=semB):
            j = 2 * i
            pltpu.make_async_copy(h_hbm.at[src_row(b, j)], bufA, semA).wait()
            pltpu.sync_copy(bufA, acc.at[dst_row(b, j)], add=True)
            pltpu.async_copy(h_hbm.at[src_row(b, j + 2)], bufA, semA)
            pltpu.make_async_copy(h_hbm.at[src_row(b, j + 1)], bufB,
                                  semB).wait()
            pltpu.sync_copy(bufB, acc.at[dst_row(b, j + 1)], add=True)
            pltpu.async_copy(h_hbm.at[src_row(b, j + 3)], bufB, semB)
            return carry

        lax.fori_loop(0, (CPASS - 5) // 2, _pair, 0)
        j = CPASS - 5
        pltpu.make_async_copy(h_hbm.at[src_row(b, j)], bufA, semA).wait()
        pltpu.sync_copy(bufA, acc.at[dst_row(b, j)], add=True)
        pltpu.async_copy(h_hbm.at[src_row(b, j + 2)], bufA, semA)
        pltpu.make_async_copy(h_hbm.at[src_row(b, j + 1)], bufB, semB).wait()
        pltpu.sync_copy(bufB, acc.at[dst_row(b, j + 1)], add=True)
        pltpu.async_copy(h_hbm.at[src_row(b, j + 3)], bufB, semB)
        pltpu.make_async_copy(h_hbm.at[src_row(b, j + 2)], bufA, semA).wait()
        pltpu.sync_copy(bufA, acc.at[dst_row(b, j + 2)], add=True)
        pltpu.async_copy(h_hbm.at[src_row(b, j + 4)], bufA, semA)
        pltpu.make_async_copy(h_hbm.at[src_row(b, j + 3)], bufB, semB).wait()
        pltpu.sync_copy(bufB, acc.at[dst_row(b, j + 3)], add=True)
        if p < NPASS - 1:
            pltpu.make_async_copy(idx_hbm.at[wid, p + 1],
                                  idx_v.at[pl.ds((1 - b) * 2 * CPASS,
                                                 2 * CPASS)], sem3).wait()
            pltpu.async_copy(h_hbm.at[src_row(1 - b, 0)], bufB, semB)
        pltpu.make_async_copy(h_hbm.at[src_row(b, j + 4)], bufA, semA).wait()
        pltpu.sync_copy(bufA, acc.at[dst_row(b, j + 4)], add=True)
        if p < NPASS - 1:
            pltpu.async_copy(h_hbm.at[src_row(1 - b, 1)], bufA, semA)
        if p < NPASS - 2:
            pltpu.async_copy(idx_hbm.at[wid, p + 2],
                             idx_v.at[pl.ds(b * 2 * CPASS, 2 * CPASS)], sem3)
    plsc.subcore_barrier()

    pltpu.sync_copy(acc.at[pl.ds(s * ROWS_PER_TILE, ROWS_PER_TILE)],
                    out_hbm.at[c, pl.ds(s * ROWS_PER_TILE, ROWS_PER_TILE)])


def _sc_scatter(h, idx4):
    mesh = plsc.VectorSubcoreMesh(core_axis_name="c", subcore_axis_name="s")
    return pl.kernel(
        _sc_body,
        out_type=jax.ShapeDtypeStruct((NC, N_PAD, D), jnp.float32),
        mesh=mesh,
        scratch_types=[
            pltpu.VMEM((4 * CPASS, CHUNK), jnp.int32),
            pltpu.VMEM((CHUNK, D), jnp.float32),
            pltpu.VMEM((CHUNK, D), jnp.float32),
            pltpu.VMEM_SHARED((N_PAD, D), jnp.float32),
            pltpu.SemaphoreType.DMA,
            pltpu.SemaphoreType.DMA,
            pltpu.SemaphoreType.DMA,
        ],
    )(h, idx4)


def kernel(x, edge_index, batch, W1, W2):
    src4 = edge_index[0].reshape(NW, NPASS, CPASS, CHUNK)
    dst4 = edge_index[1].reshape(NW, NPASS, CPASS, CHUNK)
    idx4 = jnp.concatenate([src4, dst4], axis=2)
    p = _sc_scatter(x, idx4)
    g1 = _comb_mm_relu(p, W1)
    q = _sc_scatter(g1, idx4)
    return _comb_mm_relu(q, W2)
